# baseline (speedup 1.0000x reference)
import functools

import jax
import jax.numpy as jnp
from jax import lax
from jax.experimental import pallas as pl
from jax.experimental.pallas import tpu as pltpu
from jax.experimental.pallas import tpu_sc as plsc

_NC = 2
_NS = 16
_NW = _NC * _NS
_L = 16
_CHUNK = 3200
_UNROLL = 16
_NBUF = 4


def _masked_edge_sums(x_flat, edge_index, home_vec, away_vec):
    n_nodes = x_flat.shape[0]
    n_edges = edge_index.shape[1]
    n_chunks = n_edges // _CHUNK
    k_per_w = -(-n_chunks // _NW)
    k_per_w = -(-k_per_w // _NBUF) * _NBUF

    mesh = plsc.VectorSubcoreMesh(core_axis_name="c", subcore_axis_name="s")

    @functools.partial(
        pl.kernel,
        mesh=mesh,
        compiler_params=pltpu.CompilerParams(needs_layout_passes=False),
        out_type=[
            jax.ShapeDtypeStruct((_NW, _L), jnp.float32),
            jax.ShapeDtypeStruct((_NW, _L), jnp.float32),
        ],
        scratch_types=[
            pltpu.VMEM((n_nodes,), jnp.float32),
            [pltpu.VMEM((2, _CHUNK), jnp.int32) for _ in range(_NBUF)],
            pltpu.VMEM((_L,), jnp.int32),
            pltpu.VMEM((_L,), jnp.int32),
            pltpu.VMEM((_L,), jnp.float32),
            pltpu.VMEM((_L,), jnp.float32),
            [pltpu.SemaphoreType.DMA for _ in range(_NBUF)],
        ],
    )
    def k(x_hbm, ei_hbm, hv_hbm, av_hbm, out_h, out_a,
          xtab, bufs, hvb, avb, sth, sta, sems):
        wid = lax.axis_index("s") * _NC + lax.axis_index("c")
        pltpu.sync_copy(hv_hbm, hvb)
        pltpu.sync_copy(av_hbm, avb)
        hv = hvb[...]
        av = avb[...]
        neg1 = jnp.full((_L,), -1, jnp.int32)
        zero = jnp.zeros((_L,), jnp.float32)

        def start(c, buf, sem):
            off = jnp.minimum(c, n_chunks - 1) * _CHUNK
            pltpu.make_async_copy(
                ei_hbm.at[:, pl.ds(off, _CHUNK)], buf, sem).start()

        def wait(buf, sem):
            pltpu.make_async_copy(
                ei_hbm.at[:, pl.ds(0, _CHUNK)], buf, sem).wait()

        def compute(buf, hvc, avc, carry):
            def vbody(i, carry2):
                ah, aa = carry2
                for u in range(_UNROLL):
                    o = (i * _UNROLL + u) * _L
                    s = buf[0, pl.ds(o, _L)]
                    d = buf[1, pl.ds(o, _L)]
                    vals = plsc.load_gather(xtab, [s])
                    ah = ah + jnp.where(d == hvc, vals, zero)
                    aa = aa + jnp.where(d == avc, vals, zero)
                return ah, aa

            return lax.fori_loop(0, _CHUNK // (_L * _UNROLL), vbody, carry)

        for i in range(_NBUF):
            start(wid + _NW * i, bufs[i], sems[i])

        pltpu.sync_copy(x_hbm, xtab)

        def body(j, carry):
            for i in range(_NBUF):
                kk = _NBUF * j + i
                c = wid + _NW * kk
                ok = jax.lax.broadcast_in_dim(c < n_chunks, (_L,), ())
                wait(bufs[i], sems[i])
                carry = compute(bufs[i], jnp.where(ok, hv, neg1),
                                jnp.where(ok, av, neg1), carry)

                @pl.when(kk + _NBUF < k_per_w)
                def _():
                    start(c + _NW * _NBUF, bufs[i], sems[i])

            return carry

        acc_h, acc_a = lax.fori_loop(0, k_per_w // _NBUF, body, (zero, zero))
        sth[...] = acc_h
        sta[...] = acc_a
        pltpu.sync_copy(sth, out_h.at[wid])
        pltpu.sync_copy(sta, out_a.at[wid])

    return k(x_flat, edge_index, home_vec, away_vec)


def kernel(x, edge_index, home, away, result):
    x_flat = x.reshape(-1)
    home_i = jnp.asarray(home, jnp.int32)
    away_i = jnp.asarray(away, jnp.int32)
    hv = jnp.full((_L,), home_i, dtype=jnp.int32)
    av = jnp.full((_L,), away_i, dtype=jnp.int32)
    out_h, out_a = _masked_edge_sums(x_flat, edge_index, hv, av)
    home_sum = out_h.sum()
    away_sum = out_a.sum()
    xh = x_flat[home_i]
    xa = x_flat[away_i]
    fh = jnp.where(result != 0, jax.nn.sigmoid(xh * 0.8 + home_sum), xh)
    fa = jnp.where(result != 0, jax.nn.sigmoid(xa * 0.8 + away_sum), xa)
    return jnp.sign(fh - fa).reshape(1)


# speedup vs baseline: 1.0227x; 1.0227x over previous
import functools

import jax
import jax.numpy as jnp
from jax import lax
from jax.experimental import pallas as pl
from jax.experimental.pallas import tpu as pltpu
from jax.experimental.pallas import tpu_sc as plsc

_NC = 2
_NS = 16
_NW = _NC * _NS
_L = 16
_CHUNK = 2560
_UNROLL = 16
_NBUF = 4


def _masked_edge_sums(x_flat, edge_index, home_vec, away_vec):
    n_nodes = x_flat.shape[0]
    n_edges = edge_index.shape[1]
    n_chunks = n_edges // _CHUNK
    k_per_w = -(-n_chunks // _NW)
    k_per_w = -(-k_per_w // _NBUF) * _NBUF

    mesh = plsc.VectorSubcoreMesh(core_axis_name="c", subcore_axis_name="s")

    @functools.partial(
        pl.kernel,
        mesh=mesh,
        compiler_params=pltpu.CompilerParams(needs_layout_passes=False),
        out_type=[
            jax.ShapeDtypeStruct((_NW, _L), jnp.float32),
            jax.ShapeDtypeStruct((_NW, _L), jnp.float32),
        ],
        scratch_types=[
            pltpu.VMEM((n_nodes,), jnp.float32),
            [pltpu.VMEM((2, _CHUNK), jnp.int32) for _ in range(_NBUF)],
            pltpu.VMEM((_L,), jnp.int32),
            pltpu.VMEM((_L,), jnp.int32),
            pltpu.VMEM((_L,), jnp.float32),
            pltpu.VMEM((_L,), jnp.float32),
            [pltpu.SemaphoreType.DMA for _ in range(_NBUF)],
            pltpu.VMEM_SHARED((n_nodes,), jnp.float32),
        ],
    )
    def k(x_hbm, ei_hbm, hv_hbm, av_hbm, out_h, out_a,
          xtab, bufs, hvb, avb, sth, sta, sems, shtab):
        wid = lax.axis_index("s") * _NC + lax.axis_index("c")
        pltpu.sync_copy(hv_hbm, hvb)
        pltpu.sync_copy(av_hbm, avb)
        hv = hvb[...]
        av = avb[...]
        neg1 = jnp.full((_L,), -1, jnp.int32)
        zero = jnp.zeros((_L,), jnp.float32)

        def start(c, buf, sem):
            off = jnp.minimum(c, n_chunks - 1) * _CHUNK
            pltpu.make_async_copy(
                ei_hbm.at[:, pl.ds(off, _CHUNK)], buf, sem).start()

        def wait(buf, sem):
            pltpu.make_async_copy(
                ei_hbm.at[:, pl.ds(0, _CHUNK)], buf, sem).wait()

        def compute(buf, hvc, avc, carry):
            def vbody(i, carry2):
                ah, aa = carry2
                for u in range(_UNROLL):
                    o = (i * _UNROLL + u) * _L
                    s = buf[0, pl.ds(o, _L)]
                    d = buf[1, pl.ds(o, _L)]
                    vals = plsc.load_gather(xtab, [s])
                    ah = ah + jnp.where(d == hvc, vals, zero)
                    aa = aa + jnp.where(d == avc, vals, zero)
                return ah, aa

            return lax.fori_loop(0, _CHUNK // (_L * _UNROLL), vbody, carry)

        for i in range(_NBUF):
            start(wid + _NW * i, bufs[i], sems[i])

        @pl.when(lax.axis_index("s") == 0)
        def _():
            pltpu.sync_copy(x_hbm, shtab)

        plsc.subcore_barrier()
        pltpu.sync_copy(shtab, xtab)

        def body(j, carry):
            for i in range(_NBUF):
                kk = _NBUF * j + i
                c = wid + _NW * kk
                ok = jax.lax.broadcast_in_dim(c < n_chunks, (_L,), ())
                wait(bufs[i], sems[i])
                carry = compute(bufs[i], jnp.where(ok, hv, neg1),
                                jnp.where(ok, av, neg1), carry)

                @pl.when(kk + _NBUF < k_per_w)
                def _():
                    start(c + _NW * _NBUF, bufs[i], sems[i])

            return carry

        acc_h, acc_a = lax.fori_loop(0, k_per_w // _NBUF, body, (zero, zero))
        sth[...] = acc_h
        sta[...] = acc_a
        pltpu.sync_copy(sth, out_h.at[wid])
        pltpu.sync_copy(sta, out_a.at[wid])

    return k(x_flat, edge_index, home_vec, away_vec)


def kernel(x, edge_index, home, away, result):
    x_flat = x.reshape(-1)
    home_i = jnp.asarray(home, jnp.int32)
    away_i = jnp.asarray(away, jnp.int32)
    hv = jnp.full((_L,), home_i, dtype=jnp.int32)
    av = jnp.full((_L,), away_i, dtype=jnp.int32)
    out_h, out_a = _masked_edge_sums(x_flat, edge_index, hv, av)
    home_sum = out_h.sum()
    away_sum = out_a.sum()
    xh = x_flat[home_i]
    xa = x_flat[away_i]
    fh = jnp.where(result != 0, jax.nn.sigmoid(xh * 0.8 + home_sum), xh)
    fa = jnp.where(result != 0, jax.nn.sigmoid(xa * 0.8 + away_sum), xa)
    return jnp.sign(fh - fa).reshape(1)
